# baseline (speedup 1.0000x reference)
import jax
import jax.numpy as jnp
from jax import lax
from jax.experimental import pallas as pl
from jax.experimental.pallas import tpu as pltpu
from jax.experimental.pallas import tpu_sc as plsc

N = 10000
E = 320000
IN_C = 128
HID = 128
OUT_C = 64

NC = 2
NS = 16
B = 40
K = 5
CH = K * B
EPAD = 327680
EPT = E // NS
HPT = EPT // 2
SEGS = ((0, 30), (6000, 20))
SEGMAX = 6000
NPT = N // NS


def _mm_body(x_ref, w1_ref, wo_ref, b_ref, e1_ref, e2_ref,
             ga_ref, ga2_ref, g0_ref, s1_ref, d1_ref, s2_ref, d2_ref):
    dn = (((1,), (1,)), ((), ()))
    h0 = lax.dot_general(x_ref[...], w1_ref[...], dn,
                         preferred_element_type=jnp.float32)
    wo = wo_ref[...]
    g1 = lax.dot_general(h0, wo[:, HID:2 * HID], dn,
                         preferred_element_type=jnp.float32)
    g2 = lax.dot_general(h0, wo[:, 2 * HID:3 * HID], dn,
                         preferred_element_type=jnp.float32)
    ga = jnp.concatenate([g1, g2], axis=1)
    ga_ref[...] = ga
    ga2_ref[...] = ga
    g0_ref[...] = lax.dot_general(h0, wo[:, 0:HID], dn,
                                  preferred_element_type=jnp.float32) + b_ref[...]
    s1_ref[...] = e1_ref[0, :] * 2
    d1_ref[...] = e1_ref[1, :]
    s2_ref[...] = e2_ref[0, :] * 2 + 1
    d2_ref[...] = e2_ref[1, :]


def _matmul(x, W1, W_out, b_out, e1, e2):
    R = 2000
    G = N // R
    EB = EPAD // G
    f32 = jnp.float32
    i32 = jnp.int32
    return pl.pallas_call(
        _mm_body,
        grid=(G,),
        in_specs=[
            pl.BlockSpec((R, IN_C), lambda i: (i, 0)),
            pl.BlockSpec((HID, IN_C), lambda i: (0, 0)),
            pl.BlockSpec((OUT_C, 3 * HID), lambda i: (0, 0)),
            pl.BlockSpec((1, OUT_C), lambda i: (0, 0)),
            pl.BlockSpec((2, EB), lambda i: (0, i)),
            pl.BlockSpec((2, EB), lambda i: (0, i)),
        ],
        out_specs=[
            pl.BlockSpec((R, 2 * OUT_C), lambda i: (i, 0)),
            pl.BlockSpec((R, 2 * OUT_C), lambda i: (i, 0)),
            pl.BlockSpec((R, OUT_C), lambda i: (i, 0)),
            pl.BlockSpec((EB,), lambda i: (i,)),
            pl.BlockSpec((EB,), lambda i: (i,)),
            pl.BlockSpec((EB,), lambda i: (i,)),
            pl.BlockSpec((EB,), lambda i: (i,)),
        ],
        out_shape=[
            jax.ShapeDtypeStruct((N, 2 * OUT_C), f32),
            jax.ShapeDtypeStruct((N, 2 * OUT_C), f32),
            jax.ShapeDtypeStruct((N, OUT_C), f32),
            jax.ShapeDtypeStruct((EPAD,), i32),
            jax.ShapeDtypeStruct((EPAD,), i32),
            jax.ShapeDtypeStruct((EPAD,), i32),
            jax.ShapeDtypeStruct((EPAD,), i32),
        ],
    )(x, W1, W_out, b_out.reshape(1, OUT_C), e1, e2)


def _sc_body(gaf_hbm, ga2_hbm, z_hbm, s1_hbm, d1_hbm, s2_hbm, d2_hbm, out_hbm,
             sidxh, didxh, sidxs, didxs, rows0, rows1, acc, table,
             gsem0, gsem1, ssem0, ssem1):
    cid = lax.axis_index("c")
    sid = lax.axis_index("s")
    r0 = sid * NPT

    pltpu.sync_copy(ga2_hbm.at[pl.ds(r0, NPT), pl.ds(cid * OUT_C, OUT_C)],
                    table.at[pl.ds(r0, NPT)])
    pltpu.sync_copy(z_hbm.at[pl.ds(r0, NPT)], acc.at[pl.ds(r0, NPT)])
    plsc.subcore_barrier()

    def fire_gh(c, buf, sem):
        for j in range(K):
            pltpu.async_copy(gaf_hbm.at[sidxh.at[pl.ds(c * CH + j * B, B)]],
                             buf.at[pl.ds(j * B, B)], sem)

    def fire_gs(c, buf, sem):
        for j in range(K):
            pltpu.async_copy(table.at[sidxs.at[pl.ds(c * CH + j * B, B)]],
                             buf.at[pl.ds(j * B, B)], sem)

    def fire_s(c, buf, sem, didx):
        for j in range(K):
            pltpu.async_copy(buf.at[pl.ds(j * B, B)],
                             acc.at[didx.at[pl.ds(c * CH + j * B, B)]],
                             sem, add=True)

    def drain_g(buf, sem):
        pltpu.make_async_copy(gaf_hbm.at[pl.ds(0, CH)], buf, sem).wait()

    def drain_s(buf, sem):
        pltpu.make_async_copy(buf, acc.at[pl.ds(0, CH)], sem).wait()

    def run_edges(s_hbm, d_hbm):
        for seg_off, niter in SEGS:
            tbh = sid * EPT + seg_off
            tbs = sid * EPT + HPT + seg_off
            nedge = niter * CH
            pltpu.sync_copy(s_hbm.at[pl.ds(tbh, nedge)], sidxh.at[pl.ds(0, nedge)])
            pltpu.sync_copy(d_hbm.at[pl.ds(tbh, nedge)], didxh.at[pl.ds(0, nedge)])
            pltpu.sync_copy(s_hbm.at[pl.ds(tbs, nedge)], sidxs.at[pl.ds(0, nedge)])
            pltpu.sync_copy(d_hbm.at[pl.ds(tbs, nedge)], didxs.at[pl.ds(0, nedge)])

            def fix(k, carry):
                sidxs[pl.ds(k * 16, 16)] = sidxs[pl.ds(k * 16, 16)] >> 1
                return carry

            lax.fori_loop(0, nedge // 16, fix, 0)

            fire_gh(0, rows0, gsem0)

            def body(i, carry):
                drain_g(rows0, gsem0)
                fire_s(i, rows0, ssem0, didxh)

                @pl.when(i > 0)
                def _():
                    drain_s(rows1, ssem1)

                fire_gs(i, rows1, gsem1)
                drain_g(rows1, gsem1)
                fire_s(i, rows1, ssem1, didxs)
                drain_s(rows0, ssem0)

                @pl.when(i < niter - 1)
                def _():
                    fire_gh(i + 1, rows0, gsem0)

                return carry

            lax.fori_loop(0, niter, body, 0)
            drain_s(rows1, ssem1)

    @pl.when(cid == 0)
    def _():
        run_edges(s1_hbm, d1_hbm)

    @pl.when(cid == 1)
    def _():
        run_edges(s2_hbm, d2_hbm)

    plsc.subcore_barrier()

    @pl.when(cid == 0)
    def _():
        pltpu.sync_copy(acc.at[pl.ds(r0, NPT)],
                        out_hbm.at[pl.ds(r0, NPT), pl.ds(0, OUT_C)])

    @pl.when(cid == 1)
    def _():
        pltpu.sync_copy(acc.at[pl.ds(r0, NPT)],
                        out_hbm.at[pl.ds(r0, NPT), pl.ds(OUT_C, OUT_C)])


def _scatter(gaf, ga2, s1, d1, s2, d2):
    z = jnp.zeros((N, OUT_C), jnp.float32)
    mesh = plsc.VectorSubcoreMesh(core_axis_name="c", subcore_axis_name="s",
                                  num_cores=NC, num_subcores=NS)
    f = pl.kernel(
        _sc_body,
        out_type=jax.ShapeDtypeStruct((N, 2 * OUT_C), jnp.float32),
        mesh=mesh,
        scratch_types=[
            pltpu.VMEM((SEGMAX,), jnp.int32),
            pltpu.VMEM((SEGMAX,), jnp.int32),
            pltpu.VMEM((SEGMAX,), jnp.int32),
            pltpu.VMEM((SEGMAX,), jnp.int32),
            pltpu.VMEM((CH, OUT_C), jnp.float32),
            pltpu.VMEM((CH, OUT_C), jnp.float32),
            pltpu.VMEM_SHARED((N, OUT_C), jnp.float32),
            pltpu.VMEM_SHARED((N, OUT_C), jnp.float32),
            pltpu.SemaphoreType.DMA,
            pltpu.SemaphoreType.DMA,
            pltpu.SemaphoreType.DMA,
            pltpu.SemaphoreType.DMA,
        ],
        compiler_params=pltpu.CompilerParams(use_tc_tiling_on_sc=False),
    )
    return f(gaf, ga2, z, s1, d1, s2, d2)


def _add_body(o2_ref, g0_ref, out_ref):
    o2 = o2_ref[...]
    out_ref[...] = o2[:, 0:OUT_C] + o2[:, OUT_C:2 * OUT_C] + g0_ref[...]


def _add(o2, g0b):
    R = 2000
    return pl.pallas_call(
        _add_body,
        grid=(N // R,),
        in_specs=[
            pl.BlockSpec((R, 2 * OUT_C), lambda i: (i, 0)),
            pl.BlockSpec((R, OUT_C), lambda i: (i, 0)),
        ],
        out_specs=pl.BlockSpec((R, OUT_C), lambda i: (i, 0)),
        out_shape=jax.ShapeDtypeStruct((N, OUT_C), jnp.float32),
    )(o2, g0b)


def kernel(x, edge_index1, edge_index2, W1, W_out, b_out):
    GA, GA2, g0b, s1, d1, s2, d2 = _matmul(x, W1, W_out, b_out,
                                           edge_index1, edge_index2)
    gaf = GA.reshape(2 * N, OUT_C)
    OUT = _scatter(gaf, GA2, s1, d1, s2, d2)
    return _add(OUT, g0b)


# speedup vs baseline: 1.1554x; 1.1554x over previous
import jax
import jax.numpy as jnp
from jax import lax
from jax.experimental import pallas as pl
from jax.experimental.pallas import tpu as pltpu
from jax.experimental.pallas import tpu_sc as plsc

N = 10000
E = 320000
IN_C = 128
HID = 128
OUT_C = 64

NC = 2
NS = 16
B = 80
K = 5
CH = K * B
EPAD = 327680
EPT = E // NS
SEGS = ((0, 24), (19200, 1))
SEGMAX = 19200
NPT = N // NS


def _mm_body(x_ref, w1_ref, wo_ref, b_ref, e1_ref, e2_ref,
             ga_ref, g0_ref, z_ref, s1_ref, d1_ref, s2_ref, d2_ref):
    dn = (((1,), (1,)), ((), ()))
    h0 = lax.dot_general(x_ref[...], w1_ref[...], dn,
                         preferred_element_type=jnp.float32)
    wo = wo_ref[...]
    g1 = lax.dot_general(h0, wo[:, HID:2 * HID], dn,
                         preferred_element_type=jnp.float32)
    g2 = lax.dot_general(h0, wo[:, 2 * HID:3 * HID], dn,
                         preferred_element_type=jnp.float32)
    ga_ref[...] = jnp.concatenate([g1, g2], axis=1)
    g0_ref[...] = lax.dot_general(h0, wo[:, 0:HID], dn,
                                  preferred_element_type=jnp.float32) + b_ref[...]
    z_ref[...] = jnp.zeros_like(z_ref)
    s1_ref[...] = e1_ref[0, :] * 2
    d1_ref[...] = e1_ref[1, :]
    s2_ref[...] = e2_ref[0, :] * 2 + 1
    d2_ref[...] = e2_ref[1, :]


def _matmul(x, W1, W_out, b_out, e1, e2):
    R = 2000
    G = N // R
    EB = EPAD // G
    f32 = jnp.float32
    i32 = jnp.int32
    return pl.pallas_call(
        _mm_body,
        grid=(G,),
        in_specs=[
            pl.BlockSpec((R, IN_C), lambda i: (i, 0)),
            pl.BlockSpec((HID, IN_C), lambda i: (0, 0)),
            pl.BlockSpec((OUT_C, 3 * HID), lambda i: (0, 0)),
            pl.BlockSpec((1, OUT_C), lambda i: (0, 0)),
            pl.BlockSpec((2, EB), lambda i: (0, i)),
            pl.BlockSpec((2, EB), lambda i: (0, i)),
        ],
        out_specs=[
            pl.BlockSpec((R, 2 * OUT_C), lambda i: (i, 0)),
            pl.BlockSpec((R, OUT_C), lambda i: (i, 0)),
            pl.BlockSpec((R, OUT_C), lambda i: (i, 0)),
            pl.BlockSpec((EB,), lambda i: (i,)),
            pl.BlockSpec((EB,), lambda i: (i,)),
            pl.BlockSpec((EB,), lambda i: (i,)),
            pl.BlockSpec((EB,), lambda i: (i,)),
        ],
        out_shape=[
            jax.ShapeDtypeStruct((N, 2 * OUT_C), f32),
            jax.ShapeDtypeStruct((N, OUT_C), f32),
            jax.ShapeDtypeStruct((N, OUT_C), f32),
            jax.ShapeDtypeStruct((EPAD,), i32),
            jax.ShapeDtypeStruct((EPAD,), i32),
            jax.ShapeDtypeStruct((EPAD,), i32),
            jax.ShapeDtypeStruct((EPAD,), i32),
        ],
    )(x, W1, W_out, b_out.reshape(1, OUT_C), e1, e2)


def _sc_body(gaf_hbm, z_hbm, s1_hbm, d1_hbm, s2_hbm, d2_hbm, out_hbm,
             sidx, didx, rows0, rows1, acc, gsem0, gsem1, ssem0, ssem1):
    cid = lax.axis_index("c")
    sid = lax.axis_index("s")
    r0 = sid * NPT

    pltpu.sync_copy(z_hbm.at[pl.ds(r0, NPT)], acc.at[pl.ds(r0, NPT)])
    plsc.subcore_barrier()

    def fire_g(c, buf, sem):
        for j in range(K):
            pltpu.async_copy(gaf_hbm.at[sidx.at[pl.ds(c * CH + j * B, B)]],
                             buf.at[pl.ds(j * B, B)], sem)

    def fire_s(c, buf, sem):
        for j in range(K):
            pltpu.async_copy(buf.at[pl.ds(j * B, B)],
                             acc.at[didx.at[pl.ds(c * CH + j * B, B)]],
                             sem, add=True)

    def drain_g(buf, sem):
        pltpu.make_async_copy(gaf_hbm.at[pl.ds(0, CH)], buf, sem).wait()

    def drain_s(buf, sem):
        pltpu.make_async_copy(buf, acc.at[pl.ds(0, CH)], sem).wait()

    def run_edges(s_hbm, d_hbm):
        for seg_off, niter in SEGS:
            tb = sid * EPT + seg_off
            nedge = niter * 2 * CH
            pltpu.sync_copy(s_hbm.at[pl.ds(tb, nedge)], sidx.at[pl.ds(0, nedge)])
            pltpu.sync_copy(d_hbm.at[pl.ds(tb, nedge)], didx.at[pl.ds(0, nedge)])

            fire_g(0, rows0, gsem0)

            def body(i, carry):
                c0 = 2 * i
                drain_g(rows0, gsem0)
                fire_s(c0, rows0, ssem0)

                @pl.when(i > 0)
                def _():
                    drain_s(rows1, ssem1)

                fire_g(c0 + 1, rows1, gsem1)
                drain_g(rows1, gsem1)
                fire_s(c0 + 1, rows1, ssem1)
                drain_s(rows0, ssem0)

                @pl.when(i < niter - 1)
                def _():
                    fire_g(c0 + 2, rows0, gsem0)

                return carry

            lax.fori_loop(0, niter, body, 0)
            drain_s(rows1, ssem1)

    @pl.when(cid == 0)
    def _():
        run_edges(s1_hbm, d1_hbm)

    @pl.when(cid == 1)
    def _():
        run_edges(s2_hbm, d2_hbm)

    plsc.subcore_barrier()

    @pl.when(cid == 0)
    def _():
        pltpu.sync_copy(acc.at[pl.ds(r0, NPT)],
                        out_hbm.at[pl.ds(r0, NPT), pl.ds(0, OUT_C)])

    @pl.when(cid == 1)
    def _():
        pltpu.sync_copy(acc.at[pl.ds(r0, NPT)],
                        out_hbm.at[pl.ds(r0, NPT), pl.ds(OUT_C, OUT_C)])


def _scatter(gaf, z, s1, d1, s2, d2):
    mesh = plsc.VectorSubcoreMesh(core_axis_name="c", subcore_axis_name="s",
                                  num_cores=NC, num_subcores=NS)
    f = pl.kernel(
        _sc_body,
        out_type=jax.ShapeDtypeStruct((N, 2 * OUT_C), jnp.float32),
        mesh=mesh,
        scratch_types=[
            pltpu.VMEM((SEGMAX,), jnp.int32),
            pltpu.VMEM((SEGMAX,), jnp.int32),
            pltpu.VMEM((CH, OUT_C), jnp.float32),
            pltpu.VMEM((CH, OUT_C), jnp.float32),
            pltpu.VMEM_SHARED((N, OUT_C), jnp.float32),
            pltpu.SemaphoreType.DMA,
            pltpu.SemaphoreType.DMA,
            pltpu.SemaphoreType.DMA,
            pltpu.SemaphoreType.DMA,
        ],
        compiler_params=pltpu.CompilerParams(use_tc_tiling_on_sc=False),
    )
    return f(gaf, z, s1, d1, s2, d2)


def _add_body(o2_ref, g0_ref, out_ref):
    o2 = o2_ref[...]
    out_ref[...] = o2[:, 0:OUT_C] + o2[:, OUT_C:2 * OUT_C] + g0_ref[...]


def _add(o2, g0b):
    R = 2000
    return pl.pallas_call(
        _add_body,
        grid=(N // R,),
        in_specs=[
            pl.BlockSpec((R, 2 * OUT_C), lambda i: (i, 0)),
            pl.BlockSpec((R, OUT_C), lambda i: (i, 0)),
        ],
        out_specs=pl.BlockSpec((R, OUT_C), lambda i: (i, 0)),
        out_shape=jax.ShapeDtypeStruct((N, OUT_C), jnp.float32),
    )(o2, g0b)


def kernel(x, edge_index1, edge_index2, W1, W_out, b_out):
    GA, g0b, z, s1, d1, s2, d2 = _matmul(x, W1, W_out, b_out,
                                         edge_index1, edge_index2)
    gaf = GA.reshape(2 * N, OUT_C)
    OUT = _scatter(gaf, z, s1, d1, s2, d2)
    return _add(OUT, g0b)


# speedup vs baseline: 1.1928x; 1.0324x over previous
import jax
import jax.numpy as jnp
from jax import lax
from jax.experimental import pallas as pl
from jax.experimental.pallas import tpu as pltpu
from jax.experimental.pallas import tpu_sc as plsc

N = 10000
E = 320000
IN_C = 128
HID = 128
OUT_C = 64

NC = 2
NS = 16
B = 80
K = 5
CH = K * B
EPAD = 327680
EPT = E // NS
SEGS = ((0, 24), (19200, 1))
SEGMAX = 19200
NPT = N // NS


def _mm_body(x_ref, w1_ref, wo_ref, b_ref, e1_ref, e2_ref,
             ga_ref, g0_ref, s1_ref, d1_ref, s2_ref, d2_ref):
    dn = (((1,), (1,)), ((), ()))
    h0 = lax.dot_general(x_ref[...], w1_ref[...], dn,
                         preferred_element_type=jnp.float32)
    wo = wo_ref[...]
    g1 = lax.dot_general(h0, wo[:, HID:2 * HID], dn,
                         preferred_element_type=jnp.float32)
    g2 = lax.dot_general(h0, wo[:, 2 * HID:3 * HID], dn,
                         preferred_element_type=jnp.float32)
    ga_ref[...] = jnp.concatenate([g1, g2], axis=1)
    g0_ref[...] = lax.dot_general(h0, wo[:, 0:HID], dn,
                                  preferred_element_type=jnp.float32) + b_ref[...]
    s1_ref[...] = e1_ref[0, :] * 2
    d1_ref[...] = e1_ref[1, :]
    s2_ref[...] = e2_ref[0, :] * 2 + 1
    d2_ref[...] = e2_ref[1, :]


def _matmul(x, W1, W_out, b_out, e1, e2):
    R = 2000
    G = N // R
    EB = EPAD // G
    f32 = jnp.float32
    i32 = jnp.int32
    return pl.pallas_call(
        _mm_body,
        grid=(G,),
        in_specs=[
            pl.BlockSpec((R, IN_C), lambda i: (i, 0)),
            pl.BlockSpec((HID, IN_C), lambda i: (0, 0)),
            pl.BlockSpec((OUT_C, 3 * HID), lambda i: (0, 0)),
            pl.BlockSpec((1, OUT_C), lambda i: (0, 0)),
            pl.BlockSpec((2, EB), lambda i: (0, i)),
            pl.BlockSpec((2, EB), lambda i: (0, i)),
        ],
        out_specs=[
            pl.BlockSpec((R, 2 * OUT_C), lambda i: (i, 0)),
            pl.BlockSpec((R, OUT_C), lambda i: (i, 0)),
            pl.BlockSpec((EB,), lambda i: (i,)),
            pl.BlockSpec((EB,), lambda i: (i,)),
            pl.BlockSpec((EB,), lambda i: (i,)),
            pl.BlockSpec((EB,), lambda i: (i,)),
        ],
        out_shape=[
            jax.ShapeDtypeStruct((N, 2 * OUT_C), f32),
            jax.ShapeDtypeStruct((N, OUT_C), f32),
            jax.ShapeDtypeStruct((EPAD,), i32),
            jax.ShapeDtypeStruct((EPAD,), i32),
            jax.ShapeDtypeStruct((EPAD,), i32),
            jax.ShapeDtypeStruct((EPAD,), i32),
        ],
    )(x, W1, W_out, b_out.reshape(1, OUT_C), e1, e2)


def _sc_body(gaf_hbm, z_hbm, s1_hbm, d1_hbm, s2_hbm, d2_hbm, out_hbm,
             sidx, didx, rows0, rows1, acc, gsem0, gsem1, ssem0, ssem1):
    cid = lax.axis_index("c")
    sid = lax.axis_index("s")
    r0 = sid * NPT

    pltpu.sync_copy(z_hbm.at[pl.ds(r0, NPT)], acc.at[pl.ds(r0, NPT)])
    plsc.subcore_barrier()

    def fire_g(c, buf, sem):
        for j in range(K):
            pltpu.async_copy(gaf_hbm.at[sidx.at[pl.ds(c * CH + j * B, B)]],
                             buf.at[pl.ds(j * B, B)], sem)

    def fire_s(c, buf, sem):
        for j in range(K):
            pltpu.async_copy(buf.at[pl.ds(j * B, B)],
                             acc.at[didx.at[pl.ds(c * CH + j * B, B)]],
                             sem, add=True)

    def drain_g(buf, sem):
        pltpu.make_async_copy(gaf_hbm.at[pl.ds(0, CH)], buf, sem).wait()

    def drain_s(buf, sem):
        pltpu.make_async_copy(buf, acc.at[pl.ds(0, CH)], sem).wait()

    def run_edges(s_hbm, d_hbm):
        for seg_off, niter in SEGS:
            tb = sid * EPT + seg_off
            nedge = niter * 2 * CH
            pltpu.sync_copy(s_hbm.at[pl.ds(tb, nedge)], sidx.at[pl.ds(0, nedge)])
            pltpu.sync_copy(d_hbm.at[pl.ds(tb, nedge)], didx.at[pl.ds(0, nedge)])

            fire_g(0, rows0, gsem0)

            def body(i, carry):
                c0 = 2 * i
                drain_g(rows0, gsem0)
                fire_s(c0, rows0, ssem0)

                @pl.when(i > 0)
                def _():
                    drain_s(rows1, ssem1)

                fire_g(c0 + 1, rows1, gsem1)
                drain_g(rows1, gsem1)
                fire_s(c0 + 1, rows1, ssem1)
                drain_s(rows0, ssem0)

                @pl.when(i < niter - 1)
                def _():
                    fire_g(c0 + 2, rows0, gsem0)

                return carry

            lax.fori_loop(0, niter, body, 0)
            drain_s(rows1, ssem1)

    @pl.when(cid == 0)
    def _():
        run_edges(s1_hbm, d1_hbm)

    @pl.when(cid == 1)
    def _():
        run_edges(s2_hbm, d2_hbm)

    plsc.subcore_barrier()

    @pl.when(cid == 0)
    def _():
        pltpu.sync_copy(acc.at[pl.ds(r0, NPT)],
                        out_hbm.at[pl.ds(r0, NPT), pl.ds(0, OUT_C)])

    @pl.when(cid == 1)
    def _():
        pltpu.sync_copy(acc.at[pl.ds(r0, NPT)],
                        out_hbm.at[pl.ds(r0, NPT), pl.ds(OUT_C, OUT_C)])


def _scatter(gaf, s1, d1, s2, d2):
    z = jnp.zeros((N, OUT_C), jnp.float32)
    mesh = plsc.VectorSubcoreMesh(core_axis_name="c", subcore_axis_name="s",
                                  num_cores=NC, num_subcores=NS)
    f = pl.kernel(
        _sc_body,
        out_type=jax.ShapeDtypeStruct((N, 2 * OUT_C), jnp.float32),
        mesh=mesh,
        scratch_types=[
            pltpu.VMEM((SEGMAX,), jnp.int32),
            pltpu.VMEM((SEGMAX,), jnp.int32),
            pltpu.VMEM((CH, OUT_C), jnp.float32),
            pltpu.VMEM((CH, OUT_C), jnp.float32),
            pltpu.VMEM_SHARED((N, OUT_C), jnp.float32),
            pltpu.SemaphoreType.DMA,
            pltpu.SemaphoreType.DMA,
            pltpu.SemaphoreType.DMA,
            pltpu.SemaphoreType.DMA,
        ],
        compiler_params=pltpu.CompilerParams(use_tc_tiling_on_sc=False),
    )
    return f(gaf, z, s1, d1, s2, d2)


def _add_body(o2_ref, g0_ref, out_ref):
    o2 = o2_ref[...]
    out_ref[...] = o2[:, 0:OUT_C] + o2[:, OUT_C:2 * OUT_C] + g0_ref[...]


def _add(o2, g0b):
    R = 2000
    return pl.pallas_call(
        _add_body,
        grid=(N // R,),
        in_specs=[
            pl.BlockSpec((R, 2 * OUT_C), lambda i: (i, 0)),
            pl.BlockSpec((R, OUT_C), lambda i: (i, 0)),
        ],
        out_specs=pl.BlockSpec((R, OUT_C), lambda i: (i, 0)),
        out_shape=jax.ShapeDtypeStruct((N, OUT_C), jnp.float32),
    )(o2, g0b)


def kernel(x, edge_index1, edge_index2, W1, W_out, b_out):
    GA, g0b, s1, d1, s2, d2 = _matmul(x, W1, W_out, b_out,
                                      edge_index1, edge_index2)
    gaf = GA.reshape(2 * N, OUT_C)
    OUT = _scatter(gaf, s1, d1, s2, d2)
    return _add(OUT, g0b)
